# initial kernel scaffold (unmeasured)
import jax
import jax.numpy as jnp
from jax import lax
from jax.experimental import pallas as pl
from jax.experimental.pallas import tpu as pltpu

N_DEV = 8


def kernel(x, dy):
    m, d = x.shape
    _, f = dy.shape
    d_per = d // N_DEV

    def body(x_ref, dy_ref, out_ref, comm_ref, send_sems, recv_sems):
        my = lax.axis_index("i")
        left = (my - 1) % N_DEV
        right = (my + 1) % N_DEV

        barrier_sem = pltpu.get_barrier_semaphore()
        for nbr in (left, right):
            pl.semaphore_signal(
                barrier_sem, inc=1,
                device_id=(nbr,), device_id_type=pl.DeviceIdType.MESH,
            )
        pl.semaphore_wait(barrier_sem, 2)

        def chunk(c):
            xs = x_ref[:, pl.ds(c * d_per, d_per)]
            return lax.dot_general(
                xs, dy_ref[:, :],
                dimension_numbers=(((0,), (0,)), ((), ())),
                preferred_element_type=jnp.float32,
            )

        comm_ref[0, :, :] = chunk((my - 1) % N_DEV)

        for s in range(N_DEV - 1):
            rdma = pltpu.make_async_remote_copy(
                src_ref=comm_ref.at[s],
                dst_ref=comm_ref.at[s + 1],
                send_sem=send_sems.at[s],
                recv_sem=recv_sems.at[s],
                device_id=(right,),
                device_id_type=pl.DeviceIdType.MESH,
            )
            rdma.start()
            contrib = chunk((my - s - 2) % N_DEV)
            rdma.wait()
            comm_ref[s + 1, :, :] = comm_ref[s + 1, :, :] + contrib

        out_ref[:, :] = comm_ref[N_DEV - 1, :, :]

    return pl.pallas_call(
        body,
        out_shape=jax.ShapeDtypeStruct((d_per, f), jnp.float32),
        in_specs=[
            pl.BlockSpec(memory_space=pltpu.VMEM),
            pl.BlockSpec(memory_space=pltpu.VMEM),
        ],
        out_specs=pl.BlockSpec(memory_space=pltpu.VMEM),
        scratch_shapes=[
            pltpu.VMEM((N_DEV, d_per, f), jnp.float32),
            pltpu.SemaphoreType.DMA((N_DEV - 1,)),
            pltpu.SemaphoreType.DMA((N_DEV - 1,)),
        ],
        compiler_params=pltpu.CompilerParams(collective_id=0),
    )(x, dy)


# baseline (device time: 188291 ns/iter reference)
import jax
import jax.numpy as jnp
from jax import lax
from jax.experimental import pallas as pl
from jax.experimental.pallas import tpu as pltpu

N_DEV = 8


def kernel(x, dy):
    m, d = x.shape
    _, f = dy.shape
    d_per = d // N_DEV

    def body(x_ref, dy_ref, out_ref, comm_ref, send_sems, recv_sems):
        my = lax.axis_index("i")
        left = (my - 1) % N_DEV
        right = (my + 1) % N_DEV

        barrier_sem = pltpu.get_barrier_semaphore()
        for nbr in (left, right):
            pl.semaphore_signal(
                barrier_sem, inc=1,
                device_id=(nbr,), device_id_type=pl.DeviceIdType.MESH,
            )
        pl.semaphore_wait(barrier_sem, 2)

        def chunk(c):
            xs = x_ref[:, pl.ds(c * d_per, d_per)]
            return lax.dot_general(
                xs, dy_ref[:, :],
                dimension_numbers=(((0,), (0,)), ((), ())),
                preferred_element_type=jnp.float32,
            )

        comm_ref[0, :, :] = chunk((my - 1) % N_DEV)

        for s in range(N_DEV - 1):
            rdma = pltpu.make_async_remote_copy(
                src_ref=comm_ref.at[s],
                dst_ref=comm_ref.at[s + 1],
                send_sem=send_sems.at[s],
                recv_sem=recv_sems.at[s],
                device_id=(right,),
                device_id_type=pl.DeviceIdType.MESH,
            )
            rdma.start()
            contrib = chunk((my - s - 2) % N_DEV)
            rdma.wait()
            comm_ref[s + 1, :, :] = comm_ref[s + 1, :, :] + contrib

        out_ref[:, :] = comm_ref[N_DEV - 1, :, :]

    return pl.pallas_call(
        body,
        out_shape=jax.ShapeDtypeStruct((d_per, f), jnp.float32),
        in_specs=[
            pl.BlockSpec(memory_space=pltpu.VMEM),
            pl.BlockSpec(memory_space=pltpu.VMEM),
        ],
        out_specs=pl.BlockSpec(memory_space=pltpu.VMEM),
        scratch_shapes=[
            pltpu.VMEM((N_DEV, d_per, f), jnp.float32),
            pltpu.SemaphoreType.DMA((N_DEV - 1,)),
            pltpu.SemaphoreType.DMA((N_DEV - 1,)),
        ],
        compiler_params=pltpu.CompilerParams(
            collective_id=0,
            vmem_limit_bytes=100 * 1024 * 1024,
        ),
    )(x, dy)


# device time: 112248 ns/iter; 1.6775x vs baseline; 1.6775x over previous
import jax
import jax.numpy as jnp
from jax import lax
from jax.experimental import pallas as pl
from jax.experimental.pallas import tpu as pltpu

N_DEV = 8
CW, CCW = 0, 1


def kernel(x, dy):
    m, d = x.shape
    _, f = dy.shape
    d_per = d // N_DEV
    f_half = f // 2

    def body(x_ref, dy_ref, out_ref, comm_ref, send_sems, recv_sems):
        my = lax.axis_index("i")
        left = (my - 1) % N_DEV
        right = (my + 1) % N_DEV

        barrier_sem = pltpu.get_barrier_semaphore()
        for nbr in (left, right):
            pl.semaphore_signal(
                barrier_sem, inc=1,
                device_id=(nbr,), device_id_type=pl.DeviceIdType.MESH,
            )
        pl.semaphore_wait(barrier_sem, 2)

        def chunk_half(c, half):
            xs = x_ref[:, pl.ds(c * d_per, d_per)]
            ys = dy_ref[:, pl.ds(half * f_half, f_half)]
            return lax.dot_general(
                xs, ys,
                dimension_numbers=(((0,), (0,)), ((), ())),
                preferred_element_type=jnp.float32,
            )

        comm_ref[CW, 0, :, :] = chunk_half((my - 1) % N_DEV, CW)
        comm_ref[CCW, 0, :, :] = chunk_half((my + 1) % N_DEV, CCW)

        for s in range(N_DEV - 1):
            rdma_cw = pltpu.make_async_remote_copy(
                src_ref=comm_ref.at[CW, s],
                dst_ref=comm_ref.at[CW, s + 1],
                send_sem=send_sems.at[CW, s],
                recv_sem=recv_sems.at[CW, s],
                device_id=(right,),
                device_id_type=pl.DeviceIdType.MESH,
            )
            rdma_ccw = pltpu.make_async_remote_copy(
                src_ref=comm_ref.at[CCW, s],
                dst_ref=comm_ref.at[CCW, s + 1],
                send_sem=send_sems.at[CCW, s],
                recv_sem=recv_sems.at[CCW, s],
                device_id=(left,),
                device_id_type=pl.DeviceIdType.MESH,
            )
            rdma_cw.start()
            rdma_ccw.start()
            contrib_cw = chunk_half((my - s - 2) % N_DEV, CW)
            contrib_ccw = chunk_half((my + s + 2) % N_DEV, CCW)
            rdma_cw.wait()
            comm_ref[CW, s + 1, :, :] = comm_ref[CW, s + 1, :, :] + contrib_cw
            rdma_ccw.wait()
            comm_ref[CCW, s + 1, :, :] = (
                comm_ref[CCW, s + 1, :, :] + contrib_ccw
            )

        out_ref[:, pl.ds(0, f_half)] = comm_ref[CW, N_DEV - 1, :, :]
        out_ref[:, pl.ds(f_half, f_half)] = comm_ref[CCW, N_DEV - 1, :, :]

    return pl.pallas_call(
        body,
        out_shape=jax.ShapeDtypeStruct((d_per, f), jnp.float32),
        in_specs=[
            pl.BlockSpec(memory_space=pltpu.VMEM),
            pl.BlockSpec(memory_space=pltpu.VMEM),
        ],
        out_specs=pl.BlockSpec(memory_space=pltpu.VMEM),
        scratch_shapes=[
            pltpu.VMEM((2, N_DEV, d_per, f_half), jnp.float32),
            pltpu.SemaphoreType.DMA((2, N_DEV - 1)),
            pltpu.SemaphoreType.DMA((2, N_DEV - 1)),
        ],
        compiler_params=pltpu.CompilerParams(
            collective_id=0,
            vmem_limit_bytes=100 * 1024 * 1024,
        ),
    )(x, dy)


# device time: 98459 ns/iter; 1.9124x vs baseline; 1.1400x over previous
import jax
import jax.numpy as jnp
from jax import lax
from jax.experimental import pallas as pl
from jax.experimental.pallas import tpu as pltpu

N_DEV = 8
CW, CCW = 0, 1
SUB = 2


def kernel(x, dy):
    m, d = x.shape
    _, f = dy.shape
    d_per = d // N_DEV
    f_half = f // 2
    d_sub = d_per // SUB

    def body(x_ref, dy_ref, out_ref, comm_ref, send_sems, recv_sems):
        my = lax.axis_index("i")
        left = (my - 1) % N_DEV
        right = (my + 1) % N_DEV

        barrier_sem = pltpu.get_barrier_semaphore()
        for nbr in (left, right):
            pl.semaphore_signal(
                barrier_sem, inc=1,
                device_id=(nbr,), device_id_type=pl.DeviceIdType.MESH,
            )
        pl.semaphore_wait(barrier_sem, 2)

        def contrib(c, direction):
            xs = x_ref[:, pl.ds(c * d_per, d_per)]
            ys = dy_ref[:, pl.ds(direction * f_half, f_half)]
            return lax.dot_general(
                xs, ys,
                dimension_numbers=(((0,), (0,)), ((), ())),
                preferred_element_type=jnp.float32,
            )

        def send_chunk(direction, c):
            return (my - c - 1) % N_DEV if direction == CW else (my + c + 1) % N_DEV

        def rdma(direction, s, j):
            tgt = right if direction == CW else left
            return pltpu.make_async_remote_copy(
                src_ref=comm_ref.at[direction, s, pl.ds(j * d_sub, d_sub)],
                dst_ref=comm_ref.at[direction, s + 1, pl.ds(j * d_sub, d_sub)],
                send_sem=send_sems.at[direction, s, j],
                recv_sem=recv_sems.at[direction, s, j],
                device_id=(tgt,),
                device_id_type=pl.DeviceIdType.MESH,
            )

        comm_ref[CW, 0, :, :] = contrib(send_chunk(CW, 0), CW)
        comm_ref[CCW, 0, :, :] = contrib(send_chunk(CCW, 0), CCW)
        live = {}
        for j in range(SUB):
            for direction in (CW, CCW):
                r = rdma(direction, 0, j)
                r.start()
                live[(direction, 0, j)] = r

        nxt = {
            direction: contrib(send_chunk(direction, 1), direction)
            for direction in (CW, CCW)
        }

        for s in range(N_DEV - 1):
            last = s == N_DEV - 2
            cur = nxt
            for j in range(SUB):
                for direction in (CW, CCW):
                    live.pop((direction, s, j)).wait()
                    acc = (
                        comm_ref[direction, s + 1, pl.ds(j * d_sub, d_sub), :]
                        + cur[direction][j * d_sub : (j + 1) * d_sub, :]
                    )
                    if last:
                        out_ref[
                            pl.ds(j * d_sub, d_sub),
                            pl.ds(direction * f_half, f_half),
                        ] = acc
                    else:
                        comm_ref[
                            direction, s + 1, pl.ds(j * d_sub, d_sub), :
                        ] = acc
                        r = rdma(direction, s + 1, j)
                        r.start()
                        live[(direction, s + 1, j)] = r
            if not last:
                nxt = {
                    direction: contrib(send_chunk(direction, s + 2), direction)
                    for direction in (CW, CCW)
                }

    return pl.pallas_call(
        body,
        out_shape=jax.ShapeDtypeStruct((d_per, f), jnp.float32),
        in_specs=[
            pl.BlockSpec(memory_space=pltpu.VMEM),
            pl.BlockSpec(memory_space=pltpu.VMEM),
        ],
        out_specs=pl.BlockSpec(memory_space=pltpu.VMEM),
        scratch_shapes=[
            pltpu.VMEM((2, N_DEV, d_per, f_half), jnp.float32),
            pltpu.SemaphoreType.DMA((2, N_DEV - 1, SUB)),
            pltpu.SemaphoreType.DMA((2, N_DEV - 1, SUB)),
        ],
        compiler_params=pltpu.CompilerParams(
            collective_id=0,
            vmem_limit_bytes=100 * 1024 * 1024,
        ),
    )(x, dy)


# device time: 97266 ns/iter; 1.9358x vs baseline; 1.0123x over previous
import jax
import jax.numpy as jnp
from jax import lax
from jax.experimental import pallas as pl
from jax.experimental.pallas import tpu as pltpu

N_DEV = 8
CW, CCW = 0, 1
SUB = 2
RING = (0, 1, 2, 3, 7, 6, 5, 4)


def kernel(x, dy):
    m, d = x.shape
    _, f = dy.shape
    d_per = d // N_DEV
    f_half = f // 2
    f_sub = f_half // SUB

    def body(x_ref, dy_ref, out_ref, comm_ref, send_sems, recv_sems):
        my = lax.axis_index("i")

        idx = lax.broadcasted_iota(jnp.int32, (1, N_DEV), 1)
        ring_arr = jnp.where(idx < 4, idx, 11 - idx)
        pos = jnp.sum(jnp.where(ring_arr == my, idx, 0))

        def ring_at(k):
            return jnp.sum(jnp.where(idx == (k % N_DEV), ring_arr, 0))

        right = ring_at(pos + 1)
        left = ring_at(pos - 1)

        barrier_sem = pltpu.get_barrier_semaphore()
        for nbr in (left, right):
            pl.semaphore_signal(
                barrier_sem, inc=1,
                device_id=(nbr,), device_id_type=pl.DeviceIdType.MESH,
            )
        pl.semaphore_wait(barrier_sem, 2)

        def contrib(c, direction, j):
            xs = x_ref[:, pl.ds(c * d_per, d_per)]
            ys = dy_ref[:, pl.ds(direction * f_half + j * f_sub, f_sub)]
            return lax.dot_general(
                xs, ys,
                dimension_numbers=(((0,), (0,)), ((), ())),
                preferred_element_type=jnp.float32,
            )

        def send_chunk(direction, s):
            k = pos - s - 1 if direction == CW else pos + s + 1
            return ring_at(k)

        def rdma(direction, s, j):
            tgt = right if direction == CW else left
            return pltpu.make_async_remote_copy(
                src_ref=comm_ref.at[direction, s, :, pl.ds(j * f_sub, f_sub)],
                dst_ref=comm_ref.at[
                    direction, s + 1, :, pl.ds(j * f_sub, f_sub)
                ],
                send_sem=send_sems.at[direction, s, j],
                recv_sem=recv_sems.at[direction, s, j],
                device_id=(tgt,),
                device_id_type=pl.DeviceIdType.MESH,
            )

        live = {}
        for j in range(SUB):
            for direction in (CW, CCW):
                comm_ref[direction, 0, :, pl.ds(j * f_sub, f_sub)] = contrib(
                    send_chunk(direction, 0), direction, j
                )
                r = rdma(direction, 0, j)
                r.start()
                live[(direction, 0, j)] = r

        nxt = {
            (direction, j): contrib(send_chunk(direction, 1), direction, j)
            for direction in (CW, CCW)
            for j in range(SUB)
        }

        for s in range(N_DEV - 1):
            last = s == N_DEV - 2
            cur = nxt
            for j in range(SUB):
                for direction in (CW, CCW):
                    live.pop((direction, s, j)).wait()
                    acc = (
                        comm_ref[direction, s + 1, :, pl.ds(j * f_sub, f_sub)]
                        + cur[(direction, j)]
                    )
                    if last:
                        out_ref[
                            :, pl.ds(direction * f_half + j * f_sub, f_sub)
                        ] = acc
                    else:
                        comm_ref[
                            direction, s + 1, :, pl.ds(j * f_sub, f_sub)
                        ] = acc
                        r = rdma(direction, s + 1, j)
                        r.start()
                        live[(direction, s + 1, j)] = r
            if not last:
                nxt = {
                    (direction, j): contrib(
                        send_chunk(direction, s + 2), direction, j
                    )
                    for direction in (CW, CCW)
                    for j in range(SUB)
                }

    return pl.pallas_call(
        body,
        out_shape=jax.ShapeDtypeStruct((d_per, f), jnp.float32),
        in_specs=[
            pl.BlockSpec(memory_space=pltpu.VMEM),
            pl.BlockSpec(memory_space=pltpu.VMEM),
        ],
        out_specs=pl.BlockSpec(memory_space=pltpu.VMEM),
        scratch_shapes=[
            pltpu.VMEM((2, N_DEV, d_per, f_half), jnp.float32),
            pltpu.SemaphoreType.DMA((2, N_DEV - 1, SUB)),
            pltpu.SemaphoreType.DMA((2, N_DEV - 1, SUB)),
        ],
        compiler_params=pltpu.CompilerParams(
            collective_id=0,
            vmem_limit_bytes=100 * 1024 * 1024,
        ),
    )(x, dy)
